# Initial kernel scaffold; baseline (speedup 1.0000x reference)
import jax
import jax.numpy as jnp
from jax.experimental import pallas as pl


def kernel(users, items, user_table, item_table, W, b):
    raise NotImplementedError("write your pallas kernel here")


# speedup vs baseline: 1.2278x; 1.2278x over previous
import functools

import jax
import jax.numpy as jnp
from jax import lax
from jax.experimental import pallas as pl
from jax.experimental.pallas import tpu as pltpu
from jax.experimental.pallas import tpu_sc as plsc

D = 128


@functools.lru_cache(maxsize=None)
def _gmf_kernel(B, b_per_w, ch, nc):
  n_ch = b_per_w // ch
  mesh = plsc.VectorSubcoreMesh(core_axis_name="c", subcore_axis_name="s")

  @functools.partial(
      pl.kernel,
      mesh=mesh,
      compiler_params=pltpu.CompilerParams(needs_layout_passes=False),
      out_type=jax.ShapeDtypeStruct((B,), jnp.float32),
      scratch_types=[
          pltpu.VMEM((b_per_w,), jnp.int32),
          pltpu.VMEM((b_per_w,), jnp.int32),
          pltpu.VMEM((ch, D), jnp.float32),
          pltpu.VMEM((ch, D), jnp.float32),
          pltpu.VMEM((144,), jnp.float32),
          pltpu.VMEM((ch * 16,), jnp.float32),
          pltpu.VMEM((b_per_w,), jnp.float32),
          pltpu.SemaphoreType.DMA,
          pltpu.SemaphoreType.DMA,
      ],
  )
  def k(users_hbm, items_hbm, ut_hbm, it_hbm, wb_hbm, out_hbm,
        uidx, iidx, urows, vrows, wb, accbuf, outv, sem_u, sem_v):
    wid = lax.axis_index("s") * nc + lax.axis_index("c")
    base = wid * b_per_w
    pltpu.sync_copy(users_hbm.at[pl.ds(base, b_per_w)], uidx)
    pltpu.sync_copy(items_hbm.at[pl.ds(base, b_per_w)], iidx)
    pltpu.sync_copy(wb_hbm, wb)
    wvecs = [wb[pl.ds(16 * j, 16)] for j in range(8)]
    bias_vec = wb[pl.ds(128, 16)]
    col16 = lax.iota(jnp.int32, 16) * 16

    for c in range(n_ch):
      cu = pltpu.async_copy(ut_hbm.at[uidx.at[pl.ds(c * ch, ch)]], urows, sem_u)
      cv = pltpu.async_copy(it_hbm.at[iidx.at[pl.ds(c * ch, ch)]], vrows, sem_v)
      cu.wait()
      cv.wait()

      def row_body(i, _):
        acc = urows[i, pl.ds(0, 16)] * vrows[i, pl.ds(0, 16)] * wvecs[0]
        for j in range(1, 8):
          acc = acc + urows[i, pl.ds(16 * j, 16)] * vrows[i, pl.ds(16 * j, 16)] * wvecs[j]
        accbuf[pl.ds(i * 16, 16)] = acc
        return 0

      lax.fori_loop(0, ch, row_body, 0)

      def grp_body(g, _, c=c):
        gbase = g * 256
        s = plsc.load_gather(accbuf, [col16 + gbase])
        for l in range(1, 16):
          s = s + plsc.load_gather(accbuf, [col16 + (gbase + l)])
        outv[pl.ds(c * ch + g * 16, 16)] = s + bias_vec
        return 0

      lax.fori_loop(0, ch // 16, grp_body, 0)

    pltpu.sync_copy(outv, out_hbm.at[pl.ds(base, b_per_w)])

  return k


def kernel(users, items, user_table, item_table, W, b):
  B = users.shape[0]
  wb = jnp.concatenate([W.reshape(-1), jnp.broadcast_to(b, (16,))])
  info = plsc.get_sparse_core_info()
  nw = info.num_cores * info.num_subcores
  k = _gmf_kernel(B, B // nw, 256, info.num_cores)
  return k(users.astype(jnp.int32), items.astype(jnp.int32),
           user_table, item_table, wb)


# speedup vs baseline: 1.3692x; 1.1151x over previous
import functools

import jax
import jax.numpy as jnp
from jax import lax
from jax.experimental import pallas as pl
from jax.experimental.pallas import tpu as pltpu
from jax.experimental.pallas import tpu_sc as plsc

D = 128


@functools.lru_cache(maxsize=None)
def _gmf_kernel(B, b_per_w, ch, nc):
  n_ch = b_per_w // ch
  mesh = plsc.VectorSubcoreMesh(core_axis_name="c", subcore_axis_name="s")

  @functools.partial(
      pl.kernel,
      mesh=mesh,
      compiler_params=pltpu.CompilerParams(needs_layout_passes=False),
      out_type=jax.ShapeDtypeStruct((B,), jnp.float32),
      scratch_types=[
          pltpu.VMEM((b_per_w,), jnp.int32),
          pltpu.VMEM((b_per_w,), jnp.int32),
          pltpu.VMEM((2, ch, D), jnp.float32),
          pltpu.VMEM((2, ch, D), jnp.float32),
          pltpu.VMEM((144,), jnp.float32),
          pltpu.VMEM((ch * 16,), jnp.float32),
          pltpu.VMEM((b_per_w,), jnp.float32),
          pltpu.SemaphoreType.DMA,
          pltpu.SemaphoreType.DMA,
          pltpu.SemaphoreType.DMA,
          pltpu.SemaphoreType.DMA,
      ],
  )
  def k(users_hbm, items_hbm, ut_hbm, it_hbm, wb_hbm, out_hbm,
        uidx, iidx, ubuf, vbuf, wb, accbuf, outv,
        sem_u0, sem_u1, sem_v0, sem_v1):
    wid = lax.axis_index("s") * nc + lax.axis_index("c")
    base = wid * b_per_w
    pltpu.sync_copy(users_hbm.at[pl.ds(base, b_per_w)], uidx)
    pltpu.sync_copy(items_hbm.at[pl.ds(base, b_per_w)], iidx)
    sems_u = (sem_u0, sem_u1)
    sems_v = (sem_v0, sem_v1)

    def start(c):
      p = c % 2
      hu = pltpu.async_copy(
          ut_hbm.at[uidx.at[pl.ds(c * ch, ch)]], ubuf.at[p], sems_u[p])
      hv = pltpu.async_copy(
          it_hbm.at[iidx.at[pl.ds(c * ch, ch)]], vbuf.at[p], sems_v[p])
      return hu, hv

    handles = {0: start(0)}
    pltpu.sync_copy(wb_hbm, wb)
    wvecs = [wb[pl.ds(16 * j, 16)] for j in range(8)]
    bias_vec = wb[pl.ds(128, 16)]
    col16 = lax.iota(jnp.int32, 16) * 16

    for c in range(n_ch):
      hu, hv = handles[c]
      if c + 1 < n_ch:
        handles[c + 1] = start(c + 1)
      hu.wait()
      hv.wait()
      p = c % 2
      urows = ubuf.at[p]
      vrows = vbuf.at[p]

      def row_body(i, _, urows=urows, vrows=vrows):
        acc = urows[i, pl.ds(0, 16)] * vrows[i, pl.ds(0, 16)] * wvecs[0]
        for j in range(1, 8):
          acc = acc + urows[i, pl.ds(16 * j, 16)] * vrows[i, pl.ds(16 * j, 16)] * wvecs[j]
        accbuf[pl.ds(i * 16, 16)] = acc
        return 0

      lax.fori_loop(0, ch, row_body, 0)

      def grp_body(g, _, c=c):
        gbase = g * 256
        s = plsc.load_gather(accbuf, [col16 + gbase])
        for l in range(1, 16):
          s = s + plsc.load_gather(accbuf, [col16 + (gbase + l)])
        outv[pl.ds(c * ch + g * 16, 16)] = s + bias_vec
        return 0

      lax.fori_loop(0, ch // 16, grp_body, 0)

    pltpu.sync_copy(outv, out_hbm.at[pl.ds(base, b_per_w)])

  return k


def kernel(users, items, user_table, item_table, W, b):
  B = users.shape[0]
  wb = jnp.concatenate([W.reshape(-1), jnp.broadcast_to(b, (16,))])
  info = plsc.get_sparse_core_info()
  nw = info.num_cores * info.num_subcores
  k = _gmf_kernel(B, B // nw, 128, info.num_cores)
  return k(users.astype(jnp.int32), items.astype(jnp.int32),
           user_table, item_table, wb)
